# Initial kernel scaffold; baseline (speedup 1.0000x reference)
import jax
import jax.numpy as jnp
from jax.experimental import pallas as pl


def kernel(x, target, conv1_w, conv1_b, ln1_g, ln1_b, conv2_w, conv2_b, ln2_g, ln2_b, lin_w, lin_b, emb_table):
    raise NotImplementedError("write your pallas kernel here")


# speedup vs baseline: 1.1115x; 1.1115x over previous
import functools

import jax
import jax.numpy as jnp
from jax import lax
from jax.experimental import pallas as pl
from jax.experimental.pallas import tpu as pltpu
from jax.experimental.pallas import tpu_sc as plsc

_B, _T, _C = 16, 2048, 256
_N = _B * _T
_PITCH_DIM = 256


_NC, _NS = 2, 16
_NW = _NC * _NS
_RPW = _N // _NW
_CHUNK = 128
_NCH = _RPW // _CHUNK
_LANES = 16


def _sc_body(x_hbm, tgt_hbm, tab_hbm, out_hbm, tgt_v, idx_v, rows_v, x_v,
             gsem, xsem):
    wid = lax.axis_index("s") * _NC + lax.axis_index("c")
    base = wid * _RPW

    pltpu.sync_copy(tgt_hbm.at[pl.ds(base, _RPW)], tgt_v)

    def qbody(i, carry):
        sl = pl.ds(i * _LANES, _LANES)
        s = tgt_v[sl] * float(_PITCH_DIM)
        s = jnp.minimum(jnp.maximum(s, 0.0), float(_PITCH_DIM - 1))
        idx_v[sl] = s.astype(jnp.int32)
        return carry

    lax.fori_loop(0, _RPW // _LANES, qbody, 0)

    def cbody(c, carry):
        r0 = base + c * _CHUNK
        g = pltpu.async_copy(tab_hbm.at[idx_v.at[pl.ds(c * _CHUNK, _CHUNK)]],
                             rows_v, gsem)
        xc = pltpu.async_copy(x_hbm.at[pl.ds(r0, _CHUNK)], x_v, xsem)
        g.wait()
        xc.wait()

        def arow(r, carry2):
            for j in range(_C // _LANES):
                sl = pl.ds(j * _LANES, _LANES)
                rows_v[r, sl] = rows_v[r, sl] + x_v[r, sl]
            return carry2

        lax.fori_loop(0, _CHUNK, arow, 0)
        pltpu.sync_copy(rows_v, out_hbm.at[pl.ds(r0, _CHUNK)])
        return carry

    lax.fori_loop(0, _NCH, cbody, 0)


_embed_add = functools.partial(
    pl.kernel,
    mesh=plsc.VectorSubcoreMesh(core_axis_name="c", subcore_axis_name="s"),
    out_type=jax.ShapeDtypeStruct((_N, _C), jnp.float32),
    scratch_types=[
        pltpu.VMEM((_RPW,), jnp.float32),
        pltpu.VMEM((_RPW,), jnp.int32),
        pltpu.VMEM((_CHUNK, _C), jnp.float32),
        pltpu.VMEM((_CHUNK, _C), jnp.float32),
        pltpu.SemaphoreType.DMA,
        pltpu.SemaphoreType.DMA,
    ],
)(_sc_body)


def _pred_body(x_ref, w1_ref, b1_ref, g1_ref, bb1_ref,
               w2_ref, b2_ref, g2_ref, bb2_ref, lw_ref, lb_ref, out_ref):
    xb = x_ref[0]
    rows = lax.broadcasted_iota(jnp.int32, (_T, _C), 0)

    def conv(h, w_ref, b_row):
        p0 = jnp.dot(h, w_ref[0], preferred_element_type=jnp.float32,
                     precision=lax.Precision.HIGHEST)
        p1 = jnp.dot(h, w_ref[1], preferred_element_type=jnp.float32,
                     precision=lax.Precision.HIGHEST)
        p2 = jnp.dot(h, w_ref[2], preferred_element_type=jnp.float32,
                     precision=lax.Precision.HIGHEST)
        p0r = jnp.where(rows == 0, 0.0, pltpu.roll(p0, 1, 0))
        p2r = jnp.where(rows == _T - 1, 0.0, pltpu.roll(p2, _T - 1, 0))
        return p0r + p1 + p2r + b_row

    def layernorm(h, g_row, b_row):
        mu = jnp.mean(h, axis=-1, keepdims=True)
        var = jnp.mean((h - mu) ** 2, axis=-1, keepdims=True)
        return (h - mu) * lax.rsqrt(var + 1e-5) * g_row + b_row

    h = conv(xb, w1_ref, b1_ref[...])
    h = jnp.maximum(h, 0.0)
    h = layernorm(h, g1_ref[...], bb1_ref[...])
    h = conv(h, w2_ref, b2_ref[...])
    h = jnp.maximum(h, 0.0)
    h = layernorm(h, g2_ref[...], bb2_ref[...])
    out_ref[...] = jnp.sum(h * lw_ref[...], axis=-1) + lb_ref[0, 0]


def _predict(x, w1, b1, g1, bb1, w2, b2, g2, bb2, lw, lb):
    row = pl.BlockSpec((1, _C), lambda b: (0, 0))
    return pl.pallas_call(
        _pred_body,
        grid=(_B,),
        in_specs=[
            pl.BlockSpec((1, _T, _C), lambda b: (b, 0, 0)),
            pl.BlockSpec((3, _C, _C), lambda b: (0, 0, 0)),
            row, row, row,
            pl.BlockSpec((3, _C, _C), lambda b: (0, 0, 0)),
            row, row, row,
            row,
            pl.BlockSpec((1, 1), lambda b: (0, 0)),
        ],
        out_specs=pl.BlockSpec((_T,), lambda b: (b,)),
        out_shape=jax.ShapeDtypeStruct((_N,), jnp.float32),
        compiler_params=pltpu.CompilerParams(
            dimension_semantics=("arbitrary",)),
    )(x, w1, b1, g1, bb1, w2, b2, g2, bb2, lw, lb)


def kernel(x, target, conv1_w, conv1_b, ln1_g, ln1_b, conv2_w, conv2_b,
           ln2_g, ln2_b, lin_w, lin_b, emb_table):
    x2d = x.reshape(_N, _C)
    tgt = target.reshape(_N)
    out2d = _embed_add(x2d, tgt, emb_table)
    pred = _predict(
        x, conv1_w,
        conv1_b.reshape(1, _C), ln1_g.reshape(1, _C), ln1_b.reshape(1, _C),
        conv2_w,
        conv2_b.reshape(1, _C), ln2_g.reshape(1, _C), ln2_b.reshape(1, _C),
        lin_w.reshape(1, _C), lin_b.reshape(1, 1),
    )
    return (out2d.reshape(_B, _T, _C), pred.reshape(_B, _T))


# speedup vs baseline: 2.7669x; 2.4895x over previous
import functools

import jax
import jax.numpy as jnp
from jax import lax
from jax.experimental import pallas as pl
from jax.experimental.pallas import tpu as pltpu
from jax.experimental.pallas import tpu_sc as plsc

_B, _T, _C = 16, 2048, 256
_N = _B * _T
_PITCH_DIM = 256


_NC, _NS = 2, 16
_NW = _NC * _NS
_RPW = _N // _NW
_CHUNK = 128
_NCH = _RPW // _CHUNK
_LANES = 16


def _sc_body(x_hbm, tgt_hbm, tab_hbm, out_hbm, tgt_v, idx_v, rows_v, x_v,
             gsem, xsem):
    wid = lax.axis_index("s") * _NC + lax.axis_index("c")
    base = wid * _RPW

    pltpu.sync_copy(tgt_hbm.at[pl.ds(base, _RPW)], tgt_v)

    def qbody(i, carry):
        sl = pl.ds(i * _LANES, _LANES)
        s = tgt_v[sl] * float(_PITCH_DIM)
        s = jnp.minimum(jnp.maximum(s, 0.0), float(_PITCH_DIM - 1))
        idx_v[sl] = s.astype(jnp.int32)
        return carry

    lax.fori_loop(0, _RPW // _LANES, qbody, 0)

    def cbody(c, carry):
        r0 = base + c * _CHUNK
        g = pltpu.async_copy(tab_hbm.at[idx_v.at[pl.ds(c * _CHUNK, _CHUNK)]],
                             rows_v, gsem)
        xc = pltpu.async_copy(x_hbm.at[pl.ds(r0, _CHUNK)], x_v, xsem)
        g.wait()
        xc.wait()

        def arow(r, carry2):
            for j in range(_C // _LANES):
                sl = pl.ds(j * _LANES, _LANES)
                rows_v[r, sl] = rows_v[r, sl] + x_v[r, sl]
            return carry2

        lax.fori_loop(0, _CHUNK, arow, 0)
        pltpu.sync_copy(rows_v, out_hbm.at[pl.ds(r0, _CHUNK)])
        return carry

    lax.fori_loop(0, _NCH, cbody, 0)


_embed_add = functools.partial(
    pl.kernel,
    mesh=plsc.VectorSubcoreMesh(core_axis_name="c", subcore_axis_name="s"),
    out_type=jax.ShapeDtypeStruct((_N, _C), jnp.float32),
    scratch_types=[
        pltpu.VMEM((_RPW,), jnp.float32),
        pltpu.VMEM((_RPW,), jnp.int32),
        pltpu.VMEM((_CHUNK, _C), jnp.float32),
        pltpu.VMEM((_CHUNK, _C), jnp.float32),
        pltpu.SemaphoreType.DMA,
        pltpu.SemaphoreType.DMA,
    ],
)(_sc_body)


def _pred_body(x_ref, w1_ref, b1_ref, g1_ref, bb1_ref,
               w2_ref, b2_ref, g2_ref, bb2_ref, lw_ref, lb_ref, out_ref):
    xb = x_ref[0]
    rows = lax.broadcasted_iota(jnp.int32, (_T, _C), 0)

    def conv(h, w_ref, b_row):
        p0 = jnp.dot(h, w_ref[0], preferred_element_type=jnp.float32,
                     precision=lax.Precision.DEFAULT)
        p1 = jnp.dot(h, w_ref[1], preferred_element_type=jnp.float32,
                     precision=lax.Precision.DEFAULT)
        p2 = jnp.dot(h, w_ref[2], preferred_element_type=jnp.float32,
                     precision=lax.Precision.DEFAULT)
        p0r = jnp.where(rows == 0, 0.0, pltpu.roll(p0, 1, 0))
        p2r = jnp.where(rows == _T - 1, 0.0, pltpu.roll(p2, _T - 1, 0))
        return p0r + p1 + p2r + b_row

    def layernorm(h, g_row, b_row):
        mu = jnp.mean(h, axis=-1, keepdims=True)
        var = jnp.mean((h - mu) ** 2, axis=-1, keepdims=True)
        return (h - mu) * lax.rsqrt(var + 1e-5) * g_row + b_row

    h = conv(xb, w1_ref, b1_ref[...])
    h = jnp.maximum(h, 0.0)
    h = layernorm(h, g1_ref[...], bb1_ref[...])
    h = conv(h, w2_ref, b2_ref[...])
    h = jnp.maximum(h, 0.0)
    h = layernorm(h, g2_ref[...], bb2_ref[...])
    out_ref[...] = jnp.sum(h * lw_ref[...], axis=-1) + lb_ref[0, 0]


def _predict(x, w1, b1, g1, bb1, w2, b2, g2, bb2, lw, lb):
    row = pl.BlockSpec((1, _C), lambda b: (0, 0))
    return pl.pallas_call(
        _pred_body,
        grid=(_B,),
        in_specs=[
            pl.BlockSpec((1, _T, _C), lambda b: (b, 0, 0)),
            pl.BlockSpec((3, _C, _C), lambda b: (0, 0, 0)),
            row, row, row,
            pl.BlockSpec((3, _C, _C), lambda b: (0, 0, 0)),
            row, row, row,
            row,
            pl.BlockSpec((1, 1), lambda b: (0, 0)),
        ],
        out_specs=pl.BlockSpec((_T,), lambda b: (b,)),
        out_shape=jax.ShapeDtypeStruct((_N,), jnp.float32),
        compiler_params=pltpu.CompilerParams(
            dimension_semantics=("arbitrary",)),
    )(x, w1, b1, g1, bb1, w2, b2, g2, bb2, lw, lb)


def kernel(x, target, conv1_w, conv1_b, ln1_g, ln1_b, conv2_w, conv2_b,
           ln2_g, ln2_b, lin_w, lin_b, emb_table):
    x2d = x.reshape(_N, _C)
    tgt = target.reshape(_N)
    out2d = _embed_add(x2d, tgt, emb_table)
    pred = _predict(
        x, conv1_w,
        conv1_b.reshape(1, _C), ln1_g.reshape(1, _C), ln1_b.reshape(1, _C),
        conv2_w,
        conv2_b.reshape(1, _C), ln2_g.reshape(1, _C), ln2_b.reshape(1, _C),
        lin_w.reshape(1, _C), lin_b.reshape(1, 1),
    )
    return (out2d.reshape(_B, _T, _C), pred.reshape(_B, _T))


# speedup vs baseline: 3.3778x; 1.2208x over previous
import functools

import jax
import jax.numpy as jnp
from jax import lax
from jax.experimental import pallas as pl
from jax.experimental.pallas import tpu as pltpu
from jax.experimental.pallas import tpu_sc as plsc

_B, _T, _C = 16, 2048, 256
_N = _B * _T
_PITCH_DIM = 256


_NC, _NS = 2, 16
_NW = _NC * _NS
_RPW = _N // _NW
_CHUNK = 64
_NCH = _RPW // _CHUNK
_NBUF = 3
_LANES = 16


def _sc_body(x_hbm, tgt_hbm, tab_hbm, out_hbm, tgt_v, idx_v,
             r0_v, r1_v, r2_v, x0_v, x1_v, x2_v,
             g0, g1, g2, s0, s1, s2, w0, w1, w2):
    rows = (r0_v, r1_v, r2_v)
    xvs = (x0_v, x1_v, x2_v)
    gsem = (g0, g1, g2)
    xsem = (s0, s1, s2)
    wsem = (w0, w1, w2)

    wid = lax.axis_index("s") * _NC + lax.axis_index("c")
    base = wid * _RPW

    pltpu.sync_copy(tgt_hbm.at[pl.ds(base, _RPW)], tgt_v)

    def qbody(i, carry):
        sl = pl.ds(i * _LANES, _LANES)
        s = tgt_v[sl] * float(_PITCH_DIM)
        s = jnp.minimum(jnp.maximum(s, 0.0), float(_PITCH_DIM - 1))
        idx_v[sl] = s.astype(jnp.int32)
        return carry

    lax.fori_loop(0, _RPW // _LANES, qbody, 0)

    pend_g = [None] * _NBUF
    pend_x = [None] * _NBUF
    pend_w = [None] * _NBUF

    def issue(c):
        b = c % _NBUF
        pend_g[b] = pltpu.async_copy(
            tab_hbm.at[idx_v.at[pl.ds(c * _CHUNK, _CHUNK)]], rows[b], gsem[b])
        pend_x[b] = pltpu.async_copy(
            x_hbm.at[pl.ds(base + c * _CHUNK, _CHUNK)], xvs[b], xsem[b])

    issue(0)
    issue(1)
    for c in range(_NCH):
        b = c % _NBUF
        pend_g[b].wait()
        pend_x[b].wait()

        def arow(r, carry, _rb=rows[b], _xb=xvs[b]):
            for j in range(_C // _LANES):
                sl = pl.ds(j * _LANES, _LANES)
                _rb[r, sl] = _rb[r, sl] + _xb[r, sl]
            return carry

        lax.fori_loop(0, _CHUNK, arow, 0)
        pend_w[b] = pltpu.async_copy(
            rows[b], out_hbm.at[pl.ds(base + c * _CHUNK, _CHUNK)], wsem[b])
        nc = c + 2
        if nc < _NCH:
            b2 = nc % _NBUF
            if pend_w[b2] is not None:
                pend_w[b2].wait()
                pend_w[b2] = None
            issue(nc)
    for b in range(_NBUF):
        if pend_w[b] is not None:
            pend_w[b].wait()


_embed_add = functools.partial(
    pl.kernel,
    mesh=plsc.VectorSubcoreMesh(core_axis_name="c", subcore_axis_name="s"),
    out_type=jax.ShapeDtypeStruct((_N, _C), jnp.float32),
    scratch_types=[
        pltpu.VMEM((_RPW,), jnp.float32),
        pltpu.VMEM((_RPW,), jnp.int32),
        pltpu.VMEM((_CHUNK, _C), jnp.float32),
        pltpu.VMEM((_CHUNK, _C), jnp.float32),
        pltpu.VMEM((_CHUNK, _C), jnp.float32),
        pltpu.VMEM((_CHUNK, _C), jnp.float32),
        pltpu.VMEM((_CHUNK, _C), jnp.float32),
        pltpu.VMEM((_CHUNK, _C), jnp.float32),
        pltpu.SemaphoreType.DMA,
        pltpu.SemaphoreType.DMA,
        pltpu.SemaphoreType.DMA,
        pltpu.SemaphoreType.DMA,
        pltpu.SemaphoreType.DMA,
        pltpu.SemaphoreType.DMA,
        pltpu.SemaphoreType.DMA,
        pltpu.SemaphoreType.DMA,
        pltpu.SemaphoreType.DMA,
    ],
)(_sc_body)


def _dot(a, b):
    return jnp.dot(a, b, preferred_element_type=jnp.float32,
                   precision=lax.Precision.DEFAULT)


def _pred_body(x_ref, w1_ref, b1_ref, g1_ref, bb1_ref,
               w2_ref, b2_ref, g2_ref, bb2_ref, lw_ref, lb_ref, out_ref):
    xb = x_ref[0]
    rows = lax.broadcasted_iota(jnp.int32, (_T, _C), 0)
    first = rows == 0
    last = rows == _T - 1
    mean_col = jnp.full((_C, 1), 1.0 / _C, dtype=jnp.float32)

    def conv(h, w_ref, b_row):
        p0 = _dot(h, w_ref[0])
        p1 = _dot(h, w_ref[1])
        p2 = _dot(h, w_ref[2])
        p0r = jnp.where(first, 0.0, pltpu.roll(p0, 1, 0))
        p2r = jnp.where(last, 0.0, pltpu.roll(p2, _T - 1, 0))
        return p0r + p1 + p2r + b_row

    def layernorm(h, g_row, b_row):
        mu = _dot(h, mean_col)
        msq = _dot(h * h, mean_col)
        inv = lax.rsqrt(msq - mu * mu + 1e-5)
        return (h - mu) * inv * g_row + b_row

    h = conv(xb, w1_ref, b1_ref[...])
    h = jnp.maximum(h, 0.0)
    h = layernorm(h, g1_ref[...], bb1_ref[...])
    h = conv(h, w2_ref, b2_ref[...])
    h = jnp.maximum(h, 0.0)
    h = layernorm(h, g2_ref[...], bb2_ref[...])
    out_ref[...] = _dot(h, lw_ref[...]) + lb_ref[0, 0]


def _predict(x, w1, b1, g1, bb1, w2, b2, g2, bb2, lw, lb):
    row = pl.BlockSpec((1, _C), lambda b: (0, 0))
    return pl.pallas_call(
        _pred_body,
        grid=(_B,),
        in_specs=[
            pl.BlockSpec((1, _T, _C), lambda b: (b, 0, 0)),
            pl.BlockSpec((3, _C, _C), lambda b: (0, 0, 0)),
            row, row, row,
            pl.BlockSpec((3, _C, _C), lambda b: (0, 0, 0)),
            row, row, row,
            pl.BlockSpec((_C, 1), lambda b: (0, 0)),
            pl.BlockSpec((1, 1), lambda b: (0, 0)),
        ],
        out_specs=pl.BlockSpec((_T, 1), lambda b: (b, 0)),
        out_shape=jax.ShapeDtypeStruct((_N, 1), jnp.float32),
        compiler_params=pltpu.CompilerParams(
            dimension_semantics=("arbitrary",)),
    )(x, w1, b1, g1, bb1, w2, b2, g2, bb2, lw, lb)


def kernel(x, target, conv1_w, conv1_b, ln1_g, ln1_b, conv2_w, conv2_b,
           ln2_g, ln2_b, lin_w, lin_b, emb_table):
    x2d = x.reshape(_N, _C)
    tgt = target.reshape(_N)
    out2d = _embed_add(x2d, tgt, emb_table)
    pred = _predict(
        x, conv1_w,
        conv1_b.reshape(1, _C), ln1_g.reshape(1, _C), ln1_b.reshape(1, _C),
        conv2_w,
        conv2_b.reshape(1, _C), ln2_g.reshape(1, _C), ln2_b.reshape(1, _C),
        lin_w, lin_b.reshape(1, 1),
    )
    return (out2d.reshape(_B, _T, _C), pred.reshape(_B, _T))
